# baseline (speedup 1.0000x reference)
import jax
import jax.numpy as jnp
from jax import lax
from jax.experimental import pallas as pl
from jax.experimental.pallas import tpu as pltpu

B, D, T = 32, 64, 576
K = 1024
N = B * T
BB = 8
W = BB * T
NBLK = B // BB
NC = 2
CB = BB // NC
WC = W // NC


def _vq_block(x_ref, emb_ref, embu_ref, quant_ref, idx_ref, loss_ref,
              aug_ref, acc_ref, blk_ref):
    i = pl.program_id(0)

    @pl.when(i == 0)
    def _prologue():
        embu = embu_ref[...]
        nu = jnp.sqrt(jnp.sum(embu * embu, axis=1, keepdims=True))
        kvec = lax.broadcasted_iota(jnp.int32, (K, 1), 0)
        khi = (kvec & ~7).astype(jnp.float32)
        klo = (kvec & 7).astype(jnp.float32)
        ones = jnp.ones((K, 1), jnp.float32)
        aug = jnp.concatenate(
            [emb_ref[...], nu, khi, klo, ones,
             jnp.zeros((K, 128 - D - 4), jnp.float32)], axis=1)
        aug_ref[...] = aug
        acc_ref[0] = 0.0

    def _chunk_mask(c):
        xc = jnp.concatenate(
            [x_ref[c * CB + j] for j in range(CB)], axis=1)
        nrm = jnp.sqrt(jnp.sum(xc * xc, axis=0, keepdims=True))
        xn = xc / jnp.maximum(nrm, 1e-12)
        dist = lax.dot_general(
            emb_ref[...], xn, (((1,), (0,)), ((), ())),
            preferred_element_type=jnp.float32)
        maxv = jnp.max(dist, axis=0, keepdims=True)
        return nrm, maxv, dist >= maxv

    def _emit(c, nrm, maxv, q):
        quant = q[0:D, :]
        nu_sel = q[D:D + 1, :]
        idx = (q[D + 1, :] + q[D + 2, :]).astype(jnp.int32)
        xdot = jnp.maximum(nrm, 1e-12) * maxv * nu_sel
        lrow = nrm * nrm - 2.0 * xdot + nu_sel * nu_sel
        for j in range(CB):
            quant_ref[c * CB + j] = quant[:, j * T:(j + 1) * T]
        idx_ref[0, 0, c * WC:(c + 1) * WC] = idx
        blk_ref[c] = jnp.sum(lrow)
        return jnp.max(q[D + 3, :])

    cnts = []
    for c in range(NC):
        nrm, maxv, eq = _chunk_mask(c)
        q = lax.dot_general(
            aug_ref[...], jnp.where(eq, 1.0, 0.0), (((0,), (0,)), ((), ())),
            preferred_element_type=jnp.float32)
        cnts.append(_emit(c, nrm, maxv, q))

    tie = jnp.max(jnp.stack(cnts)) > 1.5

    @pl.when(tie)
    def _exact_tiebreak():
        for c in range(NC):
            nrm, maxv, eq = _chunk_mask(c)
            kiota = lax.broadcasted_iota(jnp.int32, (K, WC), 0)
            idx_e = jnp.min(jnp.where(eq, kiota, K), axis=0)
            oh = (kiota == idx_e[None, :]).astype(jnp.float32)
            q = lax.dot_general(
                aug_ref[...], oh, (((0,), (0,)), ((), ())),
                preferred_element_type=jnp.float32)
            _emit(c, nrm, maxv, q)

    acc_ref[0] += sum(blk_ref[c] for c in range(NC))

    @pl.when(i == NBLK - 1)
    def _epilogue():
        loss_ref[0] = acc_ref[0] / (N * D)


@jax.jit
def kernel(inputs, embedding, embedding_unnormalized):
    quant, idx, loss = pl.pallas_call(
        _vq_block,
        grid=(NBLK,),
        in_specs=[
            pl.BlockSpec((BB, D, T), lambda i: (i, 0, 0)),
            pl.BlockSpec((K, D), lambda i: (0, 0)),
            pl.BlockSpec((K, D), lambda i: (0, 0)),
        ],
        out_specs=[
            pl.BlockSpec((BB, D, T), lambda i: (i, 0, 0)),
            pl.BlockSpec((1, 1, W), lambda i: (i, 0, 0)),
            pl.BlockSpec(memory_space=pltpu.SMEM),
        ],
        out_shape=[
            jax.ShapeDtypeStruct((B, D, T), jnp.float32),
            jax.ShapeDtypeStruct((NBLK, 1, W), jnp.int32),
            jax.ShapeDtypeStruct((1,), jnp.float32),
        ],
        scratch_shapes=[
            pltpu.VMEM((K, 128), jnp.float32),
            pltpu.SMEM((1,), jnp.float32),
            pltpu.SMEM((NC,), jnp.float32),
        ],
    )(inputs, embedding, embedding_unnormalized)
    loss_val = loss[0]
    return quant, loss_val, loss_val, idx.reshape(N)


# speedup vs baseline: 1.0021x; 1.0021x over previous
import jax
import jax.numpy as jnp
from jax import lax
from jax.experimental import pallas as pl
from jax.experimental.pallas import tpu as pltpu

B, D, T = 32, 64, 576
K = 1024
N = B * T
BB = 8
W = BB * T
NBLK = B // BB
NC = 2
CB = BB // NC
WC = W // NC


def _vq_block(x_ref, emb_ref, embu_ref, quant_ref, idx_ref, loss_ref,
              aug_ref, acc_ref, blk_ref):
    i = pl.program_id(0)

    @pl.when(i == 0)
    def _prologue():
        embu = embu_ref[...]
        nu = jnp.sqrt(jnp.sum(embu * embu, axis=1, keepdims=True))
        kvec = lax.broadcasted_iota(jnp.int32, (K, 1), 0)
        khi = (kvec & ~7).astype(jnp.float32)
        klo = (kvec & 7).astype(jnp.float32)
        ones = jnp.ones((K, 1), jnp.float32)
        aug = jnp.concatenate(
            [emb_ref[...], nu, khi, klo, ones,
             jnp.zeros((K, 128 - D - 4), jnp.float32)], axis=1)
        aug_ref[...] = aug.astype(jnp.bfloat16)
        acc_ref[0] = 0.0

    def _chunk_mask(c):
        xc = jnp.concatenate(
            [x_ref[c * CB + j] for j in range(CB)], axis=1)
        nrm = jnp.sqrt(jnp.sum(xc * xc, axis=0, keepdims=True))
        xn = xc / jnp.maximum(nrm, 1e-12)
        dist = lax.dot_general(
            emb_ref[...], xn, (((1,), (0,)), ((), ())),
            preferred_element_type=jnp.float32)
        maxv = jnp.max(dist, axis=0, keepdims=True)
        return nrm, maxv, dist >= maxv

    def _emit(c, nrm, maxv, q):
        quant = q[0:D, :]
        nu_sel = q[D:D + 1, :]
        idx = (q[D + 1, :] + q[D + 2, :]).astype(jnp.int32)
        xdot = jnp.maximum(nrm, 1e-12) * maxv * nu_sel
        lrow = nrm * nrm - 2.0 * xdot + nu_sel * nu_sel
        for j in range(CB):
            quant_ref[c * CB + j] = quant[:, j * T:(j + 1) * T]
        idx_ref[0, 0, c * WC:(c + 1) * WC] = idx
        blk_ref[c] = jnp.sum(lrow)
        return jnp.max(q[D + 3, :])

    cnts = []
    for c in range(NC):
        nrm, maxv, eq = _chunk_mask(c)
        q = lax.dot_general(
            aug_ref[...], eq.astype(jnp.bfloat16), (((0,), (0,)), ((), ())),
            preferred_element_type=jnp.float32)
        cnts.append(_emit(c, nrm, maxv, q))

    tie = jnp.max(jnp.stack(cnts)) > 1.5

    @pl.when(tie)
    def _exact_tiebreak():
        for c in range(NC):
            nrm, maxv, eq = _chunk_mask(c)
            kiota = lax.broadcasted_iota(jnp.int32, (K, WC), 0)
            idx_e = jnp.min(jnp.where(eq, kiota, K), axis=0)
            oh = (kiota == idx_e[None, :]).astype(jnp.bfloat16)
            q = lax.dot_general(
                aug_ref[...], oh, (((0,), (0,)), ((), ())),
                preferred_element_type=jnp.float32)
            _emit(c, nrm, maxv, q)

    acc_ref[0] += sum(blk_ref[c] for c in range(NC))

    @pl.when(i == NBLK - 1)
    def _epilogue():
        loss_ref[0] = acc_ref[0] / (N * D)


@jax.jit
def kernel(inputs, embedding, embedding_unnormalized):
    quant, idx, loss = pl.pallas_call(
        _vq_block,
        grid=(NBLK,),
        in_specs=[
            pl.BlockSpec((BB, D, T), lambda i: (i, 0, 0)),
            pl.BlockSpec((K, D), lambda i: (0, 0)),
            pl.BlockSpec((K, D), lambda i: (0, 0)),
        ],
        out_specs=[
            pl.BlockSpec((BB, D, T), lambda i: (i, 0, 0)),
            pl.BlockSpec((1, 1, W), lambda i: (i, 0, 0)),
            pl.BlockSpec(memory_space=pltpu.SMEM),
        ],
        out_shape=[
            jax.ShapeDtypeStruct((B, D, T), jnp.float32),
            jax.ShapeDtypeStruct((NBLK, 1, W), jnp.int32),
            jax.ShapeDtypeStruct((1,), jnp.float32),
        ],
        scratch_shapes=[
            pltpu.VMEM((K, 128), jnp.bfloat16),
            pltpu.SMEM((1,), jnp.float32),
            pltpu.SMEM((NC,), jnp.float32),
        ],
    )(inputs, embedding, embedding_unnormalized)
    loss_val = loss[0]
    return quant, loss_val, loss_val, idx.reshape(N)
